# baseline (speedup 1.0000x reference)
import functools

import numpy as np
import jax
import jax.numpy as jnp
from jax import lax
from jax.experimental import pallas as pl
from jax.experimental.pallas import tpu as pltpu
from jax.experimental.pallas import tpu_sc as plsc

_VOCAB = 1000000
_EMB = 64
_CTX = 2048
_B = 4
_S = 2048
_SCALE = 8.0

_NC = 2
_NS = 16
_NW = _NC * _NS
_N = _B * _S
_CHUNK = _N // _NW
_IDXV = 128
_NGATHER = _CHUNK // _IDXV
_LANES = 16


def _pos_encoding_tiled() -> np.ndarray:
    half = _EMB // 2
    positions = np.arange(_CTX)[:, np.newaxis]
    depths = np.arange(half)[np.newaxis, :] / half
    angle_rads = positions * (1.0 / (10000.0 ** depths))
    pe = np.concatenate([np.sin(angle_rads), np.cos(angle_rads)], axis=-1)
    return np.tile(pe.astype(np.float32), (_B, 1))


_PE = _pos_encoding_tiled()


def _sc_embed(x_flat, table, pe):
    mesh = plsc.VectorSubcoreMesh(core_axis_name="c", subcore_axis_name="s")

    @functools.partial(
        pl.kernel,
        mesh=mesh,
        compiler_params=pltpu.CompilerParams(use_tc_tiling_on_sc=False),
        out_type=jax.ShapeDtypeStruct((_NW, _CHUNK, _EMB), jnp.float32),
        scratch_types=[
            pltpu.VMEM((_NGATHER, _IDXV), jnp.int32),
            pltpu.VMEM((_CHUNK, _EMB), jnp.float32),
            pltpu.VMEM((_CHUNK, _EMB), jnp.float32),
            pltpu.SemaphoreType.DMA,
        ],
    )
    def k(x_hbm, table_hbm, pe_hbm, out_hbm, idx_v, rows_v, pe_v, sem):
        wid = lax.axis_index("s") * _NC + lax.axis_index("c")
        pltpu.sync_copy(x_hbm.at[wid], idx_v)
        copies = []
        for j in range(_NGATHER):
            copies.append(
                pltpu.async_copy(
                    table_hbm.at[idx_v.at[j]],
                    rows_v.at[pl.ds(j * _IDXV, _IDXV)],
                    sem,
                )
            )
        pltpu.sync_copy(pe_hbm.at[wid], pe_v)
        for c in copies:
            c.wait()

        def body(r, carry):
            for c in range(_EMB // _LANES):
                sl = pl.ds(c * _LANES, _LANES)
                rows_v[r, sl] = rows_v[r, sl] * _SCALE + pe_v[r, sl]
            return carry

        lax.fori_loop(0, _CHUNK, body, 0)
        pltpu.sync_copy(rows_v, out_hbm.at[wid])

    return k(x_flat, table, pe)


def kernel(x, table):
    x3 = x.reshape(_NW, _NGATHER, _IDXV).astype(jnp.int32)
    pe = jnp.asarray(_PE).reshape(_NW, _CHUNK, _EMB)
    out = _sc_embed(x3, table, pe)
    return out.reshape(_B, _S, _EMB)


# speedup vs baseline: 1.7008x; 1.7008x over previous
import functools

import numpy as np
import jax
import jax.numpy as jnp
from jax import lax
from jax.experimental import pallas as pl
from jax.experimental.pallas import tpu as pltpu
from jax.experimental.pallas import tpu_sc as plsc

_VOCAB = 1000000
_EMB = 64
_CTX = 2048
_B = 4
_S = 2048
_SCALE = 8.0

_NC = 2
_NS = 16
_NW = _NC * _NS
_N = _B * _S
_CHUNK = _N // _NW
_LANES = 16


def _pos_encoding_tiled() -> np.ndarray:
    half = _EMB // 2
    positions = np.arange(_CTX)[:, np.newaxis]
    depths = np.arange(half)[np.newaxis, :] / half
    angle_rads = positions * (1.0 / (10000.0 ** depths))
    pe = np.concatenate([np.sin(angle_rads), np.cos(angle_rads)], axis=-1)
    return np.tile(pe.astype(np.float32), (_B, 1))


_PE = _pos_encoding_tiled()


def _sc_embed(x_flat, table, pe):
    mesh = plsc.VectorSubcoreMesh(core_axis_name="c", subcore_axis_name="s")

    @functools.partial(
        pl.kernel,
        mesh=mesh,
        out_type=jax.ShapeDtypeStruct((_NW, _CHUNK, _EMB), jnp.float32),
        scratch_types=[
            pltpu.VMEM((_CHUNK,), jnp.int32),
            pltpu.VMEM((_CHUNK, _EMB), jnp.float32),
            pltpu.VMEM((_CHUNK, _EMB), jnp.float32),
            pltpu.SemaphoreType.DMA,
        ],
    )
    def k(x_hbm, table_hbm, pe_hbm, out_hbm, idx_v, rows_v, pe_v, sem):
        wid = lax.axis_index("s") * _NC + lax.axis_index("c")
        base = wid * _CHUNK
        pltpu.sync_copy(x_hbm.at[pl.ds(base, _CHUNK)], idx_v)

        def fire(g, carry):
            v = idx_v[pl.ds(g * _LANES, _LANES)]
            for t in range(_LANES):
                pltpu.async_copy(
                    table_hbm.at[pl.ds(v[t], 1)],
                    rows_v.at[pl.ds(g * _LANES + t, 1)],
                    sem,
                )
            return carry

        lax.fori_loop(0, _CHUNK // _LANES, fire, 0)
        pltpu.sync_copy(pe_hbm.at[wid], pe_v)
        pltpu.make_async_copy(
            table_hbm.at[pl.ds(0, _CHUNK)], rows_v, sem
        ).wait()

        def body(r, carry):
            for c in range(_EMB // _LANES):
                sl = pl.ds(c * _LANES, _LANES)
                rows_v[r, sl] = rows_v[r, sl] * _SCALE + pe_v[r, sl]
            return carry

        lax.fori_loop(0, _CHUNK, body, 0)
        pltpu.sync_copy(rows_v, out_hbm.at[wid])

    return k(x_flat, table, pe)


def kernel(x, table):
    x_flat = x.reshape(_N).astype(jnp.int32)
    pe = jnp.asarray(_PE).reshape(_NW, _CHUNK, _EMB)
    out = _sc_embed(x_flat, table, pe)
    return out.reshape(_B, _S, _EMB)


# speedup vs baseline: 4.4068x; 2.5910x over previous
import functools

import numpy as np
import jax
import jax.numpy as jnp
from jax import lax
from jax.experimental import pallas as pl
from jax.experimental.pallas import tpu as pltpu
from jax.experimental.pallas import tpu_sc as plsc

_VOCAB = 1000000
_EMB = 64
_CTX = 2048
_B = 4
_S = 2048
_SCALE = 8.0

_NC = 2
_NS = 16
_NW = _NC * _NS
_N = _B * _S
_CHUNK = _N // _NW
_LANES = 16
_NBUF = 4


def _pos_encoding() -> np.ndarray:
    half = _EMB // 2
    positions = np.arange(_CTX)[:, np.newaxis]
    depths = np.arange(half)[np.newaxis, :] / half
    angle_rads = positions * (1.0 / (10000.0 ** depths))
    pe = np.concatenate([np.sin(angle_rads), np.cos(angle_rads)], axis=-1)
    pe = np.tile(pe.astype(np.float32), (_B, 1))
    return np.ascontiguousarray(pe.reshape(_NW, _CHUNK, _EMB))


_PE = _pos_encoding()


def _sc_embed(x_flat, table_t, pe):
    mesh = plsc.VectorSubcoreMesh(core_axis_name="c", subcore_axis_name="s")

    @functools.partial(
        pl.kernel,
        mesh=mesh,
        compiler_params=pltpu.CompilerParams(needs_layout_passes=False),
        out_type=jax.ShapeDtypeStruct((_NW, _CHUNK, _EMB), jnp.float32),
        scratch_types=[
            pltpu.VMEM((_CHUNK,), jnp.int32),
            pltpu.VMEM((_CHUNK, _EMB), jnp.float32),
            pltpu.VMEM((_CHUNK, _EMB), jnp.float32),
        ]
        + [pltpu.VMEM((_EMB, 128), jnp.float32) for _ in range(_NBUF)]
        + [pltpu.SemaphoreType.DMA for _ in range(_NBUF)],
    )
    def k(x_hbm, table_hbm, pe_hbm, out_hbm, idx_v, rows_v, pe_v,
          s0, s1, s2, s3, m0, m1, m2, m3):
        slabs = (s0, s1, s2, s3)
        sems = (m0, m1, m2, m3)
        wid = lax.axis_index("s") * _NC + lax.axis_index("c")
        base = wid * _CHUNK
        pltpu.sync_copy(x_hbm.at[pl.ds(base, _CHUNK)], idx_v)
        pltpu.sync_copy(pe_hbm.at[wid], pe_v)
        lanes = lax.iota(jnp.int32, _LANES)

        def fetch(xj, b):
            start = pl.multiple_of((xj >> 7) * 128, 128)
            pltpu.async_copy(
                table_hbm.at[:, pl.ds(start, 128)], slabs[b], sems[b]
            )

        v0 = idx_v[pl.ds(0, _LANES)]
        for t in range(_NBUF):
            fetch(v0[t], t)

        def group(g, carry):
            v = idx_v[pl.ds(g * _LANES, _LANES)]
            for t in range(_LANES):
                j = g * _LANES + t
                b = t % _NBUF
                pltpu.make_async_copy(
                    table_hbm.at[:, pl.ds(0, 128)], slabs[b], sems[b]
                ).wait()
                xj = v[t]
                lane = jnp.full((_LANES,), xj & 127, jnp.int32)
                for cc in range(_EMB // _LANES):
                    col = plsc.load_gather(
                        slabs[b], [lanes + (cc * _LANES), lane]
                    )
                    sl = pl.ds(cc * _LANES, _LANES)
                    rows_v[j, sl] = col * _SCALE + pe_v[j, sl]
                if t + _NBUF < _LANES:
                    fetch(v[t + _NBUF], b)
                else:

                    @pl.when(g < _CHUNK // _LANES - 1)
                    def _():
                        vn = idx_v[pl.ds((g + 1) * _LANES, _LANES)]
                        fetch(vn[t + _NBUF - _LANES], b)

            return carry

        lax.fori_loop(0, _CHUNK // _LANES, group, 0)
        pltpu.sync_copy(rows_v, out_hbm.at[wid])

    return k(x_flat, table_t, pe)


def kernel(x, table):
    x_flat = x.reshape(_N).astype(jnp.int32)
    pe = jnp.asarray(_PE)
    out = _sc_embed(x_flat, table.T, pe)
    return out.reshape(_B, _S, _EMB)


# speedup vs baseline: 4.5839x; 1.0402x over previous
import functools

import numpy as np
import jax
import jax.numpy as jnp
from jax import lax
from jax.experimental import pallas as pl
from jax.experimental.pallas import tpu as pltpu
from jax.experimental.pallas import tpu_sc as plsc

_VOCAB = 1000000
_EMB = 64
_CTX = 2048
_B = 4
_S = 2048
_SCALE = 8.0

_NC = 2
_NS = 16
_NW = _NC * _NS
_N = _B * _S
_CHUNK = _N // _NW
_LANES = 16
_NBUF = 4


def _pos_encoding() -> np.ndarray:
    half = _EMB // 2
    positions = np.arange(_CTX)[:, np.newaxis]
    depths = np.arange(half)[np.newaxis, :] / half
    angle_rads = positions * (1.0 / (10000.0 ** depths))
    pe = np.concatenate([np.sin(angle_rads), np.cos(angle_rads)], axis=-1)
    pe = np.tile(pe.astype(np.float32), (_B, 1))
    return np.ascontiguousarray(pe.reshape(_NW, _CHUNK, _EMB))


_PE = _pos_encoding()


def _sc_embed(x_flat, table_t, pe):
    mesh = plsc.VectorSubcoreMesh(core_axis_name="c", subcore_axis_name="s")

    @functools.partial(
        pl.kernel,
        mesh=mesh,
        compiler_params=pltpu.CompilerParams(needs_layout_passes=False),
        out_type=jax.ShapeDtypeStruct((_B, _EMB, _S), jnp.float32),
        scratch_types=[
            pltpu.VMEM((_CHUNK,), jnp.int32),
            pltpu.VMEM((_EMB, _CHUNK), jnp.float32),
            pltpu.VMEM((_CHUNK, _EMB), jnp.float32),
        ]
        + [pltpu.VMEM((_EMB, 128), jnp.float32) for _ in range(_NBUF)]
        + [pltpu.SemaphoreType.DMA for _ in range(_NBUF)],
    )
    def k(x_hbm, table_hbm, pe_hbm, out_hbm, idx_v, rows_v, pe_v,
          s0, s1, s2, s3, m0, m1, m2, m3):
        slabs = (s0, s1, s2, s3)
        sems = (m0, m1, m2, m3)
        wid = lax.axis_index("s") * _NC + lax.axis_index("c")
        bb = wid >> 3
        ss = pl.multiple_of((wid & 7) * _CHUNK, _CHUNK)
        pltpu.sync_copy(x_hbm.at[bb, pl.ds(ss, _CHUNK)], idx_v)
        pltpu.sync_copy(pe_hbm.at[wid], pe_v)
        lanes = lax.iota(jnp.int32, _LANES)

        def fetch(xj, b):
            start = pl.multiple_of((xj >> 7) * 128, 128)
            pltpu.async_copy(
                table_hbm.at[:, pl.ds(start, 128)], slabs[b], sems[b]
            )

        v0 = idx_v[pl.ds(0, _LANES)]
        for t in range(_NBUF):
            fetch(v0[t], t)

        def group(g, carry):
            v = idx_v[pl.ds(g * _LANES, _LANES)]
            for t in range(_LANES):
                j = g * _LANES + t
                b = t % _NBUF
                pltpu.make_async_copy(
                    table_hbm.at[:, pl.ds(0, 128)], slabs[b], sems[b]
                ).wait()
                xj = v[t]
                lane = jnp.full((_LANES,), xj & 127, jnp.int32)
                jv = jnp.full((_LANES,), j, jnp.int32)
                for cc in range(_EMB // _LANES):
                    rowsel = lanes + (cc * _LANES)
                    col = plsc.load_gather(slabs[b], [rowsel, lane])
                    sl = pl.ds(cc * _LANES, _LANES)
                    plsc.store_scatter(
                        rows_v, [rowsel, jv], col * _SCALE + pe_v[j, sl]
                    )
                if t + _NBUF < _LANES:
                    fetch(v[t + _NBUF], b)
                else:

                    @pl.when(g < _CHUNK // _LANES - 1)
                    def _():
                        vn = idx_v[pl.ds((g + 1) * _LANES, _LANES)]
                        fetch(vn[t + _NBUF - _LANES], b)

            return carry

        lax.fori_loop(0, _CHUNK // _LANES, group, 0)
        pltpu.sync_copy(rows_v, out_hbm.at[bb, :, pl.ds(ss, _CHUNK)])

    return k(x_flat, table_t, pe)


def kernel(x, table):
    pe = jnp.asarray(_PE)
    out_t = _sc_embed(x.astype(jnp.int32), table.T, pe)
    return jnp.swapaxes(out_t, 1, 2)


# speedup vs baseline: 5.2563x; 1.1467x over previous
import functools

import numpy as np
import jax
import jax.numpy as jnp
from jax import lax
from jax.experimental import pallas as pl
from jax.experimental.pallas import tpu as pltpu
from jax.experimental.pallas import tpu_sc as plsc

_VOCAB = 1000000
_EMB = 64
_CTX = 2048
_B = 4
_S = 2048
_SCALE = 8.0

_NC = 2
_NS = 16
_NW = _NC * _NS
_N = _B * _S
_CHUNK = _N // _NW
_LANES = 16
_NBUF = 8


def _pos_encoding() -> np.ndarray:
    half = _EMB // 2
    positions = np.arange(_CTX)[:, np.newaxis]
    depths = np.arange(half)[np.newaxis, :] / half
    angle_rads = positions * (1.0 / (10000.0 ** depths))
    pe = np.concatenate([np.sin(angle_rads), np.cos(angle_rads)], axis=-1)
    pe = np.tile(pe.astype(np.float32), (_B, 1))
    return np.ascontiguousarray(pe.reshape(_NW, _CHUNK, _EMB))


_PE = _pos_encoding()


def _sc_embed(x_flat, table_t, pe):
    mesh = plsc.VectorSubcoreMesh(core_axis_name="c", subcore_axis_name="s")

    @functools.partial(
        pl.kernel,
        mesh=mesh,
        compiler_params=pltpu.CompilerParams(needs_layout_passes=False),
        out_type=jax.ShapeDtypeStruct((_B, _EMB, _S), jnp.float32),
        scratch_types=[
            pltpu.VMEM((_CHUNK,), jnp.int32),
            pltpu.VMEM((_EMB, _CHUNK), jnp.float32),
            pltpu.VMEM((_CHUNK // 2, _EMB), jnp.float32),
        ]
        + [pltpu.VMEM((_EMB, 128), jnp.float32) for _ in range(_NBUF)]
        + [pltpu.SemaphoreType.DMA for _ in range(_NBUF)],
    )
    def k(x_hbm, table_hbm, pe_hbm, out_hbm, idx_v, rows_v, pe_v,
          s0, s1, s2, s3, s4, s5, s6, s7, m0, m1, m2, m3, m4, m5, m6, m7):
        slabs = (s0, s1, s2, s3, s4, s5, s6, s7)
        sems = (m0, m1, m2, m3, m4, m5, m6, m7)
        wid = lax.axis_index("s") * _NC + lax.axis_index("c")
        bb = wid >> 3
        ss = pl.multiple_of((wid & 7) * _CHUNK, _CHUNK)
        pltpu.sync_copy(x_hbm.at[bb, pl.ds(ss, _CHUNK)], idx_v)
        lanes = lax.iota(jnp.int32, _LANES)

        def fetch(xj, b):
            start = pl.multiple_of((xj >> 7) * 128, 128)
            pltpu.async_copy(
                table_hbm.at[:, pl.ds(start, 128)], slabs[b], sems[b]
            )

        v0 = idx_v[pl.ds(0, _LANES)]
        for t in range(_NBUF):
            fetch(v0[t], t)

        def make_group(half):
            def group(g, carry):
                v = idx_v[pl.ds(g * _LANES, _LANES)]
                for t in range(_LANES):
                    j = g * _LANES + t
                    pj = j - half * (_CHUNK // 2)
                    b = t % _NBUF
                    pltpu.make_async_copy(
                        table_hbm.at[:, pl.ds(0, 128)], slabs[b], sems[b]
                    ).wait()
                    xj = v[t]
                    lane = jnp.full((_LANES,), xj & 127, jnp.int32)
                    jv = jnp.full((_LANES,), j, jnp.int32)
                    for cc in range(_EMB // _LANES):
                        rowsel = lanes + (cc * _LANES)
                        col = plsc.load_gather(slabs[b], [rowsel, lane])
                        sl = pl.ds(cc * _LANES, _LANES)
                        plsc.store_scatter(
                            rows_v, [rowsel, jv], col * _SCALE + pe_v[pj, sl]
                        )
                    if t + _NBUF < _LANES:
                        fetch(v[t + _NBUF], b)
                    else:

                        @pl.when(g < _CHUNK // _LANES - 1)
                        def _():
                            vn = idx_v[pl.ds((g + 1) * _LANES, _LANES)]
                            fetch(vn[t + _NBUF - _LANES], b)

                return carry

            return group

        half_groups = _CHUNK // _LANES // 2
        for half in range(2):
            pltpu.sync_copy(
                pe_hbm.at[wid, pl.ds(half * (_CHUNK // 2), _CHUNK // 2)], pe_v
            )
            lax.fori_loop(
                half * half_groups, (half + 1) * half_groups, make_group(half), 0
            )
        pltpu.sync_copy(rows_v, out_hbm.at[bb, :, pl.ds(ss, _CHUNK)])

    return k(x_flat, table_t, pe)


def kernel(x, table):
    pe = jnp.asarray(_PE)
    out_t = _sc_embed(x.astype(jnp.int32), table.T, pe)
    return jnp.swapaxes(out_t, 1, 2)
